# baseline (speedup 1.0000x reference)
import functools

import jax
import jax.numpy as jnp
from jax import lax
from jax.experimental import pallas as pl
from jax.experimental.pallas import tpu as pltpu
from jax.experimental.pallas import tpu_sc as plsc

_NC = 2
_NS = 16
_NW = _NC * _NS
_CK = 128


def _ceil_div(a, b):
    return (a + b - 1) // b


def _make_edge_scatter_add(n_rows, feat, nchunk, acc_rows):
    rows_per_tile = acc_rows // _NS
    assert nchunk % 2 == 0
    mesh = plsc.VectorSubcoreMesh(core_axis_name="c", subcore_axis_name="s")

    @functools.partial(
        pl.kernel,
        mesh=mesh,
        out_type=jax.ShapeDtypeStruct((_NC, acc_rows, feat), jnp.float32),
        scratch_types=[
            pltpu.VMEM((nchunk + 1, _CK), jnp.int32),
            pltpu.VMEM((nchunk + 1, _CK), jnp.int32),
            pltpu.VMEM((_CK, feat), jnp.float32),
            pltpu.VMEM((_CK, feat), jnp.float32),
            pltpu.VMEM_SHARED((acc_rows, feat), jnp.float32),
            pltpu.SemaphoreType.DMA,
            pltpu.SemaphoreType.DMA,
        ],
        compiler_params=pltpu.CompilerParams(use_tc_tiling_on_sc=False),
    )
    def k(table, src3, dst3, zrows, out, src_v, dst_v, rows0, rows1, acc,
          sem0, sem1):
        c = lax.axis_index("c")
        s = lax.axis_index("s")
        wid = s * _NC + c
        pltpu.sync_copy(zrows, acc.at[pl.ds(s * rows_per_tile, rows_per_tile)])
        pltpu.sync_copy(src3.at[wid], src_v)
        pltpu.sync_copy(dst3.at[wid], dst_v)
        plsc.subcore_barrier()

        pltpu.async_copy(table.at[src_v.at[0]], rows0, sem0)

        def pair(i, carry):
            j0 = i * 2
            pltpu.async_copy(table.at[src_v.at[j0 + 1]], rows1, sem1)
            pltpu.make_async_copy(table.at[src_v.at[j0]], rows0, sem0).wait()
            pltpu.sync_copy(rows0, acc.at[dst_v.at[j0]], add=True)
            pltpu.async_copy(table.at[src_v.at[j0 + 2]], rows0, sem0)
            pltpu.make_async_copy(
                table.at[src_v.at[j0 + 1]], rows1, sem1).wait()
            pltpu.sync_copy(rows1, acc.at[dst_v.at[j0 + 1]], add=True)
            return carry

        lax.fori_loop(0, nchunk // 2, pair, 0)
        pltpu.make_async_copy(table.at[src_v.at[nchunk]], rows0, sem0).wait()
        plsc.subcore_barrier()
        pltpu.sync_copy(
            acc.at[pl.ds(s * rows_per_tile, rows_per_tile)],
            out.at[c, pl.ds(s * rows_per_tile, rows_per_tile)],
        )

    return k


def _bn_in(x, gamma, beta):
    m = jnp.mean(x, axis=0, keepdims=True)
    v = jnp.mean((x - m) * (x - m), axis=0, keepdims=True)
    return (x - m) / jnp.sqrt(v + 1e-5) * gamma + beta


def _dot(a, b):
    return jnp.dot(a, b, preferred_element_type=jnp.float32)


def _tc1_body(feats, W1, resW1, resb1, h_o, res1_o):
    f = feats[...]
    h_o[...] = _dot(f, W1[...])
    res1_o[...] = jnp.maximum(_dot(f, resW1[...]) + resb1[...], 0.0)


def _tc2_body(n, p, res1, b1, g1, be1, resW2, resb2, x1_o, res2_o):
    agg = p[0, :n, :] + p[1, :n, :]
    conv = jnp.maximum(agg + b1[...], 0.0)
    x1 = _bn_in(conv + res1[...], g1[...], be1[...])
    x1_o[...] = x1
    res2_o[...] = jnp.maximum(_dot(x1, resW2[...]) + resb2[...], 0.0)


def _tc3_body(n, nb, npad, p, res2, W2, b2, g2, be2, aw, ab, idsc, idsr, addf,
              M1, mb1, gm, bm, M2, mb2, out_o, x2s):
    agg2 = p[0, :n, :] + p[1, :n, :]
    conv2 = jnp.maximum(_dot(agg2, W2[...]) + b2[...], 0.0)
    x2 = _bn_in(conv2 + res2[...], g2[...], be2[...])
    feat = x2.shape[1]
    z = _dot(x2, aw[...]) + ab[...]
    wgt = 1.0 / (1.0 + jnp.exp(-z))
    gcol = lax.broadcasted_iota(jnp.int32, (nb, 1), 0)
    onehot_t = (gcol == idsr[...]).astype(jnp.float32)
    hsum = _dot(onehot_t, x2 * wgt)
    x2s[0:n, :] = x2
    if npad > n:
        x2s[n:npad, :] = jnp.zeros((npad - n, feat), jnp.float32)
    blk = 128
    gids = lax.broadcasted_iota(jnp.int32, (1, nb), 1)
    rpos = lax.broadcasted_iota(jnp.int32, (blk, 1), 0)
    neg = jnp.float32(-jnp.inf)

    def step(i, hmax):
        st = i * blk
        rows = x2s[pl.ds(st, blk), :]
        idc = idsc[pl.ds(st, blk), :]
        for s in (1, 2, 4, 8, 16, 32, 64):
            rsh = jnp.concatenate(
                [jnp.full((s, feat), neg), rows[: blk - s]], axis=0)
            ish = jnp.concatenate(
                [jnp.full((s, 1), -1, jnp.int32), idc[: blk - s]], axis=0)
            rows = jnp.where(idc == ish, jnp.maximum(rows, rsh), rows)
        idn = jnp.concatenate(
            [idc[1:], jnp.full((1, 1), -1, jnp.int32)], axis=0)
        is_end = (idc != idn) | (rpos == blk - 1)
        sel = ((idc == gids) & is_end).astype(jnp.float32)
        csum = lax.dot_general(sel, rows, (((0,), (0,)), ((), ())),
                               preferred_element_type=jnp.float32)
        cnt = lax.dot_general(sel, jnp.ones((blk, 1), jnp.float32),
                              (((0,), (0,)), ((), ())),
                              preferred_element_type=jnp.float32)
        cmax = jnp.where(cnt > 0.0, csum, neg)
        return jnp.maximum(hmax, cmax)

    hmax = lax.fori_loop(0, npad // blk, step,
                         jnp.full((nb, feat), neg, jnp.float32))
    gfeat = jnp.concatenate([hsum, hmax, addf[...]], axis=1)
    hmlp = jnp.maximum(_dot(gfeat, M1[...]) + mb1[...], 0.0)
    hmlp = _bn_in(hmlp, gm[...], bm[...])
    out_o[...] = _dot(hmlp, M2[...]) + mb2[...]


def kernel(feats, edge_index, node_to_graph, add_feats, W1, b1, resW1, resb1,
           g1, be1, W2, b2, resW2, resb2, g2, be2, aw, ab, M1, mb1, gm, bm,
           M2, mb2):
    n, d = feats.shape
    h = W1.shape[1]
    nb = add_feats.shape[0]
    e = edge_index.shape[1]

    nchunk = 2 * _ceil_div(e, _NW * _CK * 2)
    e_pad = _NW * _CK * nchunk
    acc_rows = _ceil_div(n + 1, _NS * 8) * _NS * 8
    rows_per_tile = acc_rows // _NS

    src, dst = edge_index[0], edge_index[1]
    src3 = jnp.concatenate(
        [src, jnp.zeros((e_pad - e,), jnp.int32)]).reshape(_NW, nchunk, _CK)
    src3 = jnp.concatenate(
        [src3, jnp.zeros((_NW, 1, _CK), jnp.int32)], axis=1)
    dst3 = jnp.concatenate(
        [dst, jnp.full((e_pad - e,), n, jnp.int32)]).reshape(_NW, nchunk, _CK)
    dst3 = jnp.concatenate(
        [dst3, jnp.full((_NW, 1, _CK), n, jnp.int32)], axis=1)
    zrows = jnp.zeros((rows_per_tile, h), jnp.float32)

    scatter = _make_edge_scatter_add(n, h, nchunk, acc_rows)

    r = lambda v: v.reshape(1, -1)
    npad = _ceil_div(n, 128) * 128
    idsc = jnp.concatenate(
        [node_to_graph, jnp.full((npad - n,), -1, jnp.int32)]).reshape(npad, 1)
    idsr = node_to_graph.reshape(1, n)

    h1, res1 = pl.pallas_call(
        _tc1_body,
        out_shape=[jax.ShapeDtypeStruct((n, h), jnp.float32),
                   jax.ShapeDtypeStruct((n, h), jnp.float32)],
    )(feats, W1, resW1, r(resb1))

    p1 = scatter(h1, src3, dst3, zrows)

    x1, res2 = pl.pallas_call(
        functools.partial(_tc2_body, n),
        out_shape=[jax.ShapeDtypeStruct((n, h), jnp.float32),
                   jax.ShapeDtypeStruct((n, h), jnp.float32)],
    )(p1, res1, r(b1), r(g1), r(be1), resW2, r(resb2))

    p2 = scatter(x1, src3, dst3, zrows)

    out = pl.pallas_call(
        functools.partial(_tc3_body, n, nb, npad),
        out_shape=jax.ShapeDtypeStruct((nb, M2.shape[1]), jnp.float32),
        scratch_shapes=[pltpu.VMEM((npad, h), jnp.float32)],
    )(p2, res2, W2, r(b2), r(g2), r(be2), aw, r(ab), idsc, idsr, add_feats,
      M1, r(mb1), r(gm), r(bm), M2, r(mb2))
    return out


# speedup vs baseline: 1.1101x; 1.1101x over previous
import functools

import jax
import jax.numpy as jnp
from jax import lax
from jax.experimental import pallas as pl
from jax.experimental.pallas import tpu as pltpu
from jax.experimental.pallas import tpu_sc as plsc

_NC = 2
_NS = 16
_NW = _NC * _NS
_CK = 512


def _ceil_div(a, b):
    return (a + b - 1) // b


def _make_edge_scatter_add(n_rows, feat, nchunk, acc_rows):
    rows_per_tile = acc_rows // _NS
    assert nchunk % 2 == 0
    mesh = plsc.VectorSubcoreMesh(core_axis_name="c", subcore_axis_name="s")

    @functools.partial(
        pl.kernel,
        mesh=mesh,
        out_type=jax.ShapeDtypeStruct((_NC, acc_rows, feat), jnp.float32),
        scratch_types=[
            pltpu.VMEM((nchunk + 1, _CK), jnp.int32),
            pltpu.VMEM((nchunk + 1, _CK), jnp.int32),
            pltpu.VMEM((_CK, feat), jnp.float32),
            pltpu.VMEM((_CK, feat), jnp.float32),
            pltpu.VMEM_SHARED((acc_rows, feat), jnp.float32),
            pltpu.SemaphoreType.DMA,
            pltpu.SemaphoreType.DMA,
        ],
        compiler_params=pltpu.CompilerParams(use_tc_tiling_on_sc=False),
    )
    def k(table, src3, dst3, zrows, out, src_v, dst_v, rows0, rows1, acc,
          sem0, sem1):
        c = lax.axis_index("c")
        s = lax.axis_index("s")
        wid = s * _NC + c
        pltpu.sync_copy(zrows, acc.at[pl.ds(s * rows_per_tile, rows_per_tile)])
        pltpu.sync_copy(src3.at[wid], src_v)
        pltpu.sync_copy(dst3.at[wid], dst_v)
        plsc.subcore_barrier()

        def chunk(j, carry):
            pltpu.async_copy(table.at[src_v.at[j]], rows0, sem0).wait()
            pltpu.sync_copy(rows0, acc.at[dst_v.at[j]], add=True)
            return carry

        lax.fori_loop(0, nchunk, chunk, 0)
        plsc.subcore_barrier()
        pltpu.sync_copy(
            acc.at[pl.ds(s * rows_per_tile, rows_per_tile)],
            out.at[c, pl.ds(s * rows_per_tile, rows_per_tile)],
        )

    return k


def _bn_in(x, gamma, beta):
    m = jnp.mean(x, axis=0, keepdims=True)
    v = jnp.mean((x - m) * (x - m), axis=0, keepdims=True)
    return (x - m) / jnp.sqrt(v + 1e-5) * gamma + beta


def _dot(a, b):
    return jnp.dot(a, b, preferred_element_type=jnp.float32)


def _tc1_body(feats, W1, resW1, resb1, h_o, res1_o):
    f = feats[...]
    h_o[...] = _dot(f, W1[...])
    res1_o[...] = jnp.maximum(_dot(f, resW1[...]) + resb1[...], 0.0)


def _tc2_body(n, p, res1, b1, g1, be1, resW2, resb2, x1_o, res2_o):
    agg = p[0, :n, :] + p[1, :n, :]
    conv = jnp.maximum(agg + b1[...], 0.0)
    x1 = _bn_in(conv + res1[...], g1[...], be1[...])
    x1_o[...] = x1
    res2_o[...] = jnp.maximum(_dot(x1, resW2[...]) + resb2[...], 0.0)


def _tc3_body(n, nb, npad, p, res2, W2, b2, g2, be2, aw, ab, idsc, idsr, addf,
              M1, mb1, gm, bm, M2, mb2, out_o, x2s):
    agg2 = p[0, :n, :] + p[1, :n, :]
    conv2 = jnp.maximum(_dot(agg2, W2[...]) + b2[...], 0.0)
    x2 = _bn_in(conv2 + res2[...], g2[...], be2[...])
    feat = x2.shape[1]
    z = _dot(x2, aw[...]) + ab[...]
    wgt = 1.0 / (1.0 + jnp.exp(-z))
    gcol = lax.broadcasted_iota(jnp.int32, (nb, 1), 0)
    onehot_t = (gcol == idsr[...]).astype(jnp.float32)
    hsum = _dot(onehot_t, x2 * wgt)
    x2s[0:n, :] = x2
    if npad > n:
        x2s[n:npad, :] = jnp.zeros((npad - n, feat), jnp.float32)
    blk = 128
    gids = lax.broadcasted_iota(jnp.int32, (1, nb), 1)
    rpos = lax.broadcasted_iota(jnp.int32, (blk, 1), 0)
    neg = jnp.float32(-jnp.inf)

    def step(i, hmax):
        st = i * blk
        rows = x2s[pl.ds(st, blk), :]
        idc = idsc[pl.ds(st, blk), :]
        for s in (1, 2, 4, 8, 16, 32, 64):
            rsh = jnp.concatenate(
                [jnp.full((s, feat), neg), rows[: blk - s]], axis=0)
            ish = jnp.concatenate(
                [jnp.full((s, 1), -1, jnp.int32), idc[: blk - s]], axis=0)
            rows = jnp.where(idc == ish, jnp.maximum(rows, rsh), rows)
        idn = jnp.concatenate(
            [idc[1:], jnp.full((1, 1), -1, jnp.int32)], axis=0)
        is_end = (idc != idn) | (rpos == blk - 1)
        sel = ((idc == gids) & is_end).astype(jnp.float32)
        csum = lax.dot_general(sel, rows, (((0,), (0,)), ((), ())),
                               preferred_element_type=jnp.float32)
        cnt = lax.dot_general(sel, jnp.ones((blk, 1), jnp.float32),
                              (((0,), (0,)), ((), ())),
                              preferred_element_type=jnp.float32)
        cmax = jnp.where(cnt > 0.0, csum, neg)
        return jnp.maximum(hmax, cmax)

    hmax = lax.fori_loop(0, npad // blk, step,
                         jnp.full((nb, feat), neg, jnp.float32))
    gfeat = jnp.concatenate([hsum, hmax, addf[...]], axis=1)
    hmlp = jnp.maximum(_dot(gfeat, M1[...]) + mb1[...], 0.0)
    hmlp = _bn_in(hmlp, gm[...], bm[...])
    out_o[...] = _dot(hmlp, M2[...]) + mb2[...]


def kernel(feats, edge_index, node_to_graph, add_feats, W1, b1, resW1, resb1,
           g1, be1, W2, b2, resW2, resb2, g2, be2, aw, ab, M1, mb1, gm, bm,
           M2, mb2):
    n, d = feats.shape
    h = W1.shape[1]
    nb = add_feats.shape[0]
    e = edge_index.shape[1]

    nchunk = 2 * _ceil_div(e, _NW * _CK * 2)
    e_pad = _NW * _CK * nchunk
    acc_rows = _ceil_div(n + 1, _NS * 8) * _NS * 8
    rows_per_tile = acc_rows // _NS

    src, dst = edge_index[0], edge_index[1]
    src3 = jnp.concatenate(
        [src, jnp.zeros((e_pad - e,), jnp.int32)]).reshape(_NW, nchunk, _CK)
    src3 = jnp.concatenate(
        [src3, jnp.zeros((_NW, 1, _CK), jnp.int32)], axis=1)
    dst3 = jnp.concatenate(
        [dst, jnp.full((e_pad - e,), n, jnp.int32)]).reshape(_NW, nchunk, _CK)
    dst3 = jnp.concatenate(
        [dst3, jnp.full((_NW, 1, _CK), n, jnp.int32)], axis=1)
    zrows = jnp.zeros((rows_per_tile, h), jnp.float32)

    scatter = _make_edge_scatter_add(n, h, nchunk, acc_rows)

    r = lambda v: v.reshape(1, -1)
    npad = _ceil_div(n, 128) * 128
    idsc = jnp.concatenate(
        [node_to_graph, jnp.full((npad - n,), -1, jnp.int32)]).reshape(npad, 1)
    idsr = node_to_graph.reshape(1, n)

    h1, res1 = pl.pallas_call(
        _tc1_body,
        out_shape=[jax.ShapeDtypeStruct((n, h), jnp.float32),
                   jax.ShapeDtypeStruct((n, h), jnp.float32)],
    )(feats, W1, resW1, r(resb1))

    p1 = scatter(h1, src3, dst3, zrows)

    x1, res2 = pl.pallas_call(
        functools.partial(_tc2_body, n),
        out_shape=[jax.ShapeDtypeStruct((n, h), jnp.float32),
                   jax.ShapeDtypeStruct((n, h), jnp.float32)],
    )(p1, res1, r(b1), r(g1), r(be1), resW2, r(resb2))

    p2 = scatter(x1, src3, dst3, zrows)

    out = pl.pallas_call(
        functools.partial(_tc3_body, n, nb, npad),
        out_shape=jax.ShapeDtypeStruct((nb, M2.shape[1]), jnp.float32),
        scratch_shapes=[pltpu.VMEM((npad, h), jnp.float32)],
    )(p2, res2, W2, r(b2), r(g2), r(be2), aw, r(ab), idsc, idsr, add_feats,
      M1, r(mb1), r(gm), r(bm), M2, r(mb2))
    return out


# speedup vs baseline: 1.2521x; 1.1278x over previous
import functools

import jax
import jax.numpy as jnp
from jax import lax
from jax.experimental import pallas as pl
from jax.experimental.pallas import tpu as pltpu
from jax.experimental.pallas import tpu_sc as plsc

_NC = 2
_NS = 16
_NW = _NC * _NS
_CK = 64


def _ceil_div(a, b):
    return (a + b - 1) // b


def _make_edge_scatter_add(n_rows, feat, nchunk, acc_rows):
    rows_per_tile = acc_rows // _NS
    assert nchunk % 2 == 0
    mesh = plsc.VectorSubcoreMesh(core_axis_name="c", subcore_axis_name="s")

    @functools.partial(
        pl.kernel,
        mesh=mesh,
        out_type=jax.ShapeDtypeStruct((_NC, acc_rows, feat), jnp.float32),
        scratch_types=[
            pltpu.VMEM((nchunk + 1, _CK), jnp.int32),
            pltpu.VMEM((nchunk + 1, _CK), jnp.int32),
            pltpu.VMEM((_CK, feat), jnp.float32),
            pltpu.VMEM((_CK, feat), jnp.float32),
            pltpu.VMEM_SHARED((acc_rows, feat), jnp.float32),
            pltpu.SemaphoreType.DMA,
            pltpu.SemaphoreType.DMA,
        ],
        compiler_params=pltpu.CompilerParams(use_tc_tiling_on_sc=False),
    )
    def k(table, src3, dst3, zrows, out, src_v, dst_v, rows0, rows1, acc,
          sem0, sem1):
        c = lax.axis_index("c")
        s = lax.axis_index("s")
        wid = s * _NC + c
        pltpu.sync_copy(zrows, acc.at[pl.ds(s * rows_per_tile, rows_per_tile)])
        pltpu.sync_copy(src3.at[wid], src_v)
        pltpu.sync_copy(dst3.at[wid], dst_v)
        plsc.subcore_barrier()

        def chunk(j, carry):
            pltpu.async_copy(table.at[src_v.at[j]], rows0, sem0).wait()
            pltpu.sync_copy(rows0, acc.at[dst_v.at[j]], add=True)
            return carry

        lax.fori_loop(0, nchunk, chunk, 0)
        plsc.subcore_barrier()
        pltpu.sync_copy(
            acc.at[pl.ds(s * rows_per_tile, rows_per_tile)],
            out.at[c, pl.ds(s * rows_per_tile, rows_per_tile)],
        )

    return k


def _bn_in(x, gamma, beta):
    m = jnp.mean(x, axis=0, keepdims=True)
    v = jnp.mean((x - m) * (x - m), axis=0, keepdims=True)
    return (x - m) / jnp.sqrt(v + 1e-5) * gamma + beta


def _dot(a, b):
    return jnp.dot(a, b, preferred_element_type=jnp.float32)


def _tc1_body(feats, W1, resW1, resb1, h_o, res1_o):
    f = feats[...]
    h_o[...] = _dot(f, W1[...])
    res1_o[...] = jnp.maximum(_dot(f, resW1[...]) + resb1[...], 0.0)


def _tc2_body(n, p, res1, b1, g1, be1, resW2, resb2, x1_o, res2_o):
    agg = p[0, :n, :] + p[1, :n, :]
    conv = jnp.maximum(agg + b1[...], 0.0)
    x1 = _bn_in(conv + res1[...], g1[...], be1[...])
    x1_o[...] = x1
    res2_o[...] = jnp.maximum(_dot(x1, resW2[...]) + resb2[...], 0.0)


def _tc3_body(n, nb, npad, p, res2, W2, b2, g2, be2, aw, ab, idsc, idsr, addf,
              M1, mb1, gm, bm, M2, mb2, out_o, x2s):
    agg2 = p[0, :n, :] + p[1, :n, :]
    conv2 = jnp.maximum(_dot(agg2, W2[...]) + b2[...], 0.0)
    x2 = _bn_in(conv2 + res2[...], g2[...], be2[...])
    feat = x2.shape[1]
    z = _dot(x2, aw[...]) + ab[...]
    wgt = 1.0 / (1.0 + jnp.exp(-z))
    gcol = lax.broadcasted_iota(jnp.int32, (nb, 1), 0)
    onehot_t = (gcol == idsr[...]).astype(jnp.float32)
    hsum = _dot(onehot_t, x2 * wgt)
    x2s[0:n, :] = x2
    if npad > n:
        x2s[n:npad, :] = jnp.zeros((npad - n, feat), jnp.float32)
    blk = 128
    gids = lax.broadcasted_iota(jnp.int32, (1, nb), 1)
    rpos = lax.broadcasted_iota(jnp.int32, (blk, 1), 0)
    neg = jnp.float32(-jnp.inf)

    def step(i, hmax):
        st = i * blk
        rows = x2s[pl.ds(st, blk), :]
        idc = idsc[pl.ds(st, blk), :]
        for s in (1, 2, 4, 8, 16, 32, 64):
            rsh = jnp.concatenate(
                [jnp.full((s, feat), neg), rows[: blk - s]], axis=0)
            ish = jnp.concatenate(
                [jnp.full((s, 1), -1, jnp.int32), idc[: blk - s]], axis=0)
            rows = jnp.where(idc == ish, jnp.maximum(rows, rsh), rows)
        idn = jnp.concatenate(
            [idc[1:], jnp.full((1, 1), -1, jnp.int32)], axis=0)
        is_end = (idc != idn) | (rpos == blk - 1)
        sel = ((idc == gids) & is_end).astype(jnp.float32)
        csum = lax.dot_general(sel, rows, (((0,), (0,)), ((), ())),
                               preferred_element_type=jnp.float32)
        cnt = lax.dot_general(sel, jnp.ones((blk, 1), jnp.float32),
                              (((0,), (0,)), ((), ())),
                              preferred_element_type=jnp.float32)
        cmax = jnp.where(cnt > 0.0, csum, neg)
        return jnp.maximum(hmax, cmax)

    hmax = lax.fori_loop(0, npad // blk, step,
                         jnp.full((nb, feat), neg, jnp.float32))
    gfeat = jnp.concatenate([hsum, hmax, addf[...]], axis=1)
    hmlp = jnp.maximum(_dot(gfeat, M1[...]) + mb1[...], 0.0)
    hmlp = _bn_in(hmlp, gm[...], bm[...])
    out_o[...] = _dot(hmlp, M2[...]) + mb2[...]


def kernel(feats, edge_index, node_to_graph, add_feats, W1, b1, resW1, resb1,
           g1, be1, W2, b2, resW2, resb2, g2, be2, aw, ab, M1, mb1, gm, bm,
           M2, mb2):
    n, d = feats.shape
    h = W1.shape[1]
    nb = add_feats.shape[0]
    e = edge_index.shape[1]

    nchunk = 2 * _ceil_div(e, _NW * _CK * 2)
    e_pad = _NW * _CK * nchunk
    acc_rows = _ceil_div(n + 1, _NS * 8) * _NS * 8
    rows_per_tile = acc_rows // _NS

    src, dst = edge_index[0], edge_index[1]
    src3 = jnp.concatenate(
        [src, jnp.zeros((e_pad - e,), jnp.int32)]).reshape(_NW, nchunk, _CK)
    src3 = jnp.concatenate(
        [src3, jnp.zeros((_NW, 1, _CK), jnp.int32)], axis=1)
    dst3 = jnp.concatenate(
        [dst, jnp.full((e_pad - e,), n, jnp.int32)]).reshape(_NW, nchunk, _CK)
    dst3 = jnp.concatenate(
        [dst3, jnp.full((_NW, 1, _CK), n, jnp.int32)], axis=1)
    zrows = jnp.zeros((rows_per_tile, h), jnp.float32)

    scatter = _make_edge_scatter_add(n, h, nchunk, acc_rows)

    r = lambda v: v.reshape(1, -1)
    npad = _ceil_div(n, 128) * 128
    idsc = jnp.concatenate(
        [node_to_graph, jnp.full((npad - n,), -1, jnp.int32)]).reshape(npad, 1)
    idsr = node_to_graph.reshape(1, n)

    h1, res1 = pl.pallas_call(
        _tc1_body,
        out_shape=[jax.ShapeDtypeStruct((n, h), jnp.float32),
                   jax.ShapeDtypeStruct((n, h), jnp.float32)],
    )(feats, W1, resW1, r(resb1))

    p1 = scatter(h1, src3, dst3, zrows)

    x1, res2 = pl.pallas_call(
        functools.partial(_tc2_body, n),
        out_shape=[jax.ShapeDtypeStruct((n, h), jnp.float32),
                   jax.ShapeDtypeStruct((n, h), jnp.float32)],
    )(p1, res1, r(b1), r(g1), r(be1), resW2, r(resb2))

    p2 = scatter(x1, src3, dst3, zrows)

    out = pl.pallas_call(
        functools.partial(_tc3_body, n, nb, npad),
        out_shape=jax.ShapeDtypeStruct((nb, M2.shape[1]), jnp.float32),
        scratch_shapes=[pltpu.VMEM((npad, h), jnp.float32)],
    )(p2, res2, W2, r(b2), r(g2), r(be2), aw, r(ab), idsc, idsr, add_feats,
      M1, r(mb1), r(gm), r(bm), M2, r(mb2))
    return out


# speedup vs baseline: 2.1114x; 1.6863x over previous
import functools

import jax
import jax.numpy as jnp
from jax import lax
from jax.experimental import pallas as pl
from jax.experimental.pallas import tpu as pltpu
from jax.experimental.pallas import tpu_sc as plsc

_NC = 2
_NS = 16
_NW = _NC * _NS
_CK = 128


def _ceil_div(a, b):
    return (a + b - 1) // b


def _make_edge_scatter_add(n_rows, feat, nchunk, acc_rows):
    rows_per_tile = acc_rows // _NS
    assert nchunk % 2 == 0
    mesh = plsc.VectorSubcoreMesh(core_axis_name="c", subcore_axis_name="s")

    @functools.partial(
        pl.kernel,
        mesh=mesh,
        out_type=jax.ShapeDtypeStruct((_NC, acc_rows, feat), jnp.float32),
        scratch_types=[
            pltpu.VMEM((nchunk + 1, _CK), jnp.int32),
            pltpu.VMEM((nchunk + 1, _CK), jnp.int32),
            pltpu.VMEM((_CK, feat), jnp.float32),
            pltpu.VMEM((_CK, feat), jnp.float32),
            pltpu.VMEM_SHARED((acc_rows, feat), jnp.float32),
            pltpu.VMEM_SHARED((acc_rows, feat), jnp.float32),
            pltpu.SemaphoreType.DMA,
            pltpu.SemaphoreType.DMA,
        ],
        compiler_params=pltpu.CompilerParams(use_tc_tiling_on_sc=False),
    )
    def k(table, src3, dst3, zrows, out, src_v, dst_v, rows0, rows1, acc,
          tsp, sem0, sem1):
        c = lax.axis_index("c")
        s = lax.axis_index("s")
        wid = s * _NC + c
        pltpu.sync_copy(zrows, acc.at[pl.ds(s * rows_per_tile, rows_per_tile)])
        pltpu.sync_copy(table.at[pl.ds(s * rows_per_tile, rows_per_tile)],
                        tsp.at[pl.ds(s * rows_per_tile, rows_per_tile)])
        pltpu.sync_copy(src3.at[wid], src_v)
        pltpu.sync_copy(dst3.at[wid], dst_v)
        plsc.subcore_barrier()

        def chunk(j, carry):
            pltpu.async_copy(tsp.at[src_v.at[j]], rows0, sem0).wait()
            pltpu.sync_copy(rows0, acc.at[dst_v.at[j]], add=True)
            return carry

        lax.fori_loop(0, nchunk, chunk, 0)
        plsc.subcore_barrier()
        pltpu.sync_copy(
            acc.at[pl.ds(s * rows_per_tile, rows_per_tile)],
            out.at[c, pl.ds(s * rows_per_tile, rows_per_tile)],
        )

    return k


def _bn_in(x, gamma, beta):
    m = jnp.mean(x, axis=0, keepdims=True)
    v = jnp.mean((x - m) * (x - m), axis=0, keepdims=True)
    return (x - m) / jnp.sqrt(v + 1e-5) * gamma + beta


def _dot(a, b):
    return jnp.dot(a, b, preferred_element_type=jnp.float32)


def _tc1_body(n, feats, W1, resW1, resb1, h_o, res1_o):
    f = feats[...]
    h_o[0:n, :] = _dot(f, W1[...])
    res1_o[...] = jnp.maximum(_dot(f, resW1[...]) + resb1[...], 0.0)


def _tc2_body(n, acc_rows, p, res1, b1, g1, be1, resW2, resb2, x1_o, res2_o):
    agg = p[0, :n, :] + p[1, :n, :]
    conv = jnp.maximum(agg + b1[...], 0.0)
    x1 = _bn_in(conv + res1[...], g1[...], be1[...])
    x1_o[0:n, :] = x1
    res2_o[...] = jnp.maximum(_dot(x1, resW2[...]) + resb2[...], 0.0)


def _tc3_body(n, nb, npad, p, res2, W2, b2, g2, be2, aw, ab, idsc, idsr, addf,
              M1, mb1, gm, bm, M2, mb2, out_o, x2s):
    agg2 = p[0, :n, :] + p[1, :n, :]
    conv2 = jnp.maximum(_dot(agg2, W2[...]) + b2[...], 0.0)
    x2 = _bn_in(conv2 + res2[...], g2[...], be2[...])
    feat = x2.shape[1]
    z = _dot(x2, aw[...]) + ab[...]
    wgt = 1.0 / (1.0 + jnp.exp(-z))
    gcol = lax.broadcasted_iota(jnp.int32, (nb, 1), 0)
    onehot_t = (gcol == idsr[...]).astype(jnp.float32)
    hsum = _dot(onehot_t, x2 * wgt)
    x2s[0:n, :] = x2
    if npad > n:
        x2s[n:npad, :] = jnp.zeros((npad - n, feat), jnp.float32)
    blk = 128
    gids = lax.broadcasted_iota(jnp.int32, (1, nb), 1)
    rpos = lax.broadcasted_iota(jnp.int32, (blk, 1), 0)
    neg = jnp.float32(-jnp.inf)

    def step(i, hmax):
        st = i * blk
        rows = x2s[pl.ds(st, blk), :]
        idc = idsc[pl.ds(st, blk), :]
        for s in (1, 2, 4, 8, 16, 32, 64):
            rsh = jnp.concatenate(
                [jnp.full((s, feat), neg), rows[: blk - s]], axis=0)
            ish = jnp.concatenate(
                [jnp.full((s, 1), -1, jnp.int32), idc[: blk - s]], axis=0)
            rows = jnp.where(idc == ish, jnp.maximum(rows, rsh), rows)
        idn = jnp.concatenate(
            [idc[1:], jnp.full((1, 1), -1, jnp.int32)], axis=0)
        is_end = (idc != idn) | (rpos == blk - 1)
        sel = ((idc == gids) & is_end).astype(jnp.float32)
        csum = lax.dot_general(sel, rows, (((0,), (0,)), ((), ())),
                               preferred_element_type=jnp.float32)
        cnt = lax.dot_general(sel, jnp.ones((blk, 1), jnp.float32),
                              (((0,), (0,)), ((), ())),
                              preferred_element_type=jnp.float32)
        cmax = jnp.where(cnt > 0.0, csum, neg)
        return jnp.maximum(hmax, cmax)

    hmax = lax.fori_loop(0, npad // blk, step,
                         jnp.full((nb, feat), neg, jnp.float32))
    gfeat = jnp.concatenate([hsum, hmax, addf[...]], axis=1)
    hmlp = jnp.maximum(_dot(gfeat, M1[...]) + mb1[...], 0.0)
    hmlp = _bn_in(hmlp, gm[...], bm[...])
    out_o[...] = _dot(hmlp, M2[...]) + mb2[...]


def kernel(feats, edge_index, node_to_graph, add_feats, W1, b1, resW1, resb1,
           g1, be1, W2, b2, resW2, resb2, g2, be2, aw, ab, M1, mb1, gm, bm,
           M2, mb2):
    n, d = feats.shape
    h = W1.shape[1]
    nb = add_feats.shape[0]
    e = edge_index.shape[1]

    nchunk = 2 * _ceil_div(e, _NW * _CK * 2)
    e_pad = _NW * _CK * nchunk
    acc_rows = _ceil_div(n + 1, _NS * 8) * _NS * 8
    rows_per_tile = acc_rows // _NS

    src, dst = edge_index[0], edge_index[1]
    src3 = jnp.concatenate(
        [src, jnp.zeros((e_pad - e,), jnp.int32)]).reshape(_NW, nchunk, _CK)
    src3 = jnp.concatenate(
        [src3, jnp.zeros((_NW, 1, _CK), jnp.int32)], axis=1)
    dst3 = jnp.concatenate(
        [dst, jnp.full((e_pad - e,), n, jnp.int32)]).reshape(_NW, nchunk, _CK)
    dst3 = jnp.concatenate(
        [dst3, jnp.full((_NW, 1, _CK), n, jnp.int32)], axis=1)
    zrows = jnp.zeros((rows_per_tile, h), jnp.float32)

    scatter = _make_edge_scatter_add(n, h, nchunk, acc_rows)

    r = lambda v: v.reshape(1, -1)
    npad = _ceil_div(n, 128) * 128
    idsc = jnp.concatenate(
        [node_to_graph, jnp.full((npad - n,), -1, jnp.int32)]).reshape(npad, 1)
    idsr = node_to_graph.reshape(1, n)

    h1, res1 = pl.pallas_call(
        functools.partial(_tc1_body, n),
        out_shape=[jax.ShapeDtypeStruct((acc_rows, h), jnp.float32),
                   jax.ShapeDtypeStruct((n, h), jnp.float32)],
    )(feats, W1, resW1, r(resb1))

    p1 = scatter(h1, src3, dst3, zrows)

    x1, res2 = pl.pallas_call(
        functools.partial(_tc2_body, n, acc_rows),
        out_shape=[jax.ShapeDtypeStruct((acc_rows, h), jnp.float32),
                   jax.ShapeDtypeStruct((n, h), jnp.float32)],
    )(p1, res1, r(b1), r(g1), r(be1), resW2, r(resb2))

    p2 = scatter(x1, src3, dst3, zrows)

    out = pl.pallas_call(
        functools.partial(_tc3_body, n, nb, npad),
        out_shape=jax.ShapeDtypeStruct((nb, M2.shape[1]), jnp.float32),
        scratch_shapes=[pltpu.VMEM((npad, h), jnp.float32)],
    )(p2, res2, W2, r(b2), r(g2), r(be2), aw, r(ab), idsc, idsr, add_feats,
      M1, r(mb1), r(gm), r(bm), M2, r(mb2))
    return out


# speedup vs baseline: 2.1258x; 1.0069x over previous
import functools

import jax
import jax.numpy as jnp
from jax import lax
from jax.experimental import pallas as pl
from jax.experimental.pallas import tpu as pltpu
from jax.experimental.pallas import tpu_sc as plsc

_NC = 2
_NS = 16
_NW = _NC * _NS
_CK = 256


def _ceil_div(a, b):
    return (a + b - 1) // b


def _make_edge_scatter_add(n_rows, feat, nchunk, acc_rows):
    rows_per_tile = acc_rows // _NS
    assert nchunk % 2 == 0
    mesh = plsc.VectorSubcoreMesh(core_axis_name="c", subcore_axis_name="s")

    @functools.partial(
        pl.kernel,
        mesh=mesh,
        out_type=jax.ShapeDtypeStruct((_NC, acc_rows, feat), jnp.float32),
        scratch_types=[
            pltpu.VMEM((nchunk + 1, _CK), jnp.int32),
            pltpu.VMEM((nchunk + 1, _CK), jnp.int32),
            pltpu.VMEM((_CK, feat), jnp.float32),
            pltpu.VMEM((_CK, feat), jnp.float32),
            pltpu.VMEM_SHARED((acc_rows, feat), jnp.float32),
            pltpu.VMEM_SHARED((acc_rows, feat), jnp.float32),
            pltpu.SemaphoreType.DMA,
            pltpu.SemaphoreType.DMA,
        ],
        compiler_params=pltpu.CompilerParams(use_tc_tiling_on_sc=False),
    )
    def k(table, src3, dst3, zrows, out, src_v, dst_v, rows0, rows1, acc,
          tsp, sem0, sem1):
        c = lax.axis_index("c")
        s = lax.axis_index("s")
        wid = s * _NC + c
        pltpu.sync_copy(zrows, acc.at[pl.ds(s * rows_per_tile, rows_per_tile)])
        pltpu.sync_copy(table.at[pl.ds(s * rows_per_tile, rows_per_tile)],
                        tsp.at[pl.ds(s * rows_per_tile, rows_per_tile)])
        pltpu.sync_copy(src3.at[wid], src_v)
        pltpu.sync_copy(dst3.at[wid], dst_v)
        plsc.subcore_barrier()

        def chunk(j, carry):
            pltpu.async_copy(tsp.at[src_v.at[j]], rows0, sem0).wait()
            pltpu.sync_copy(rows0, acc.at[dst_v.at[j]], add=True)
            return carry

        lax.fori_loop(0, nchunk, chunk, 0)
        plsc.subcore_barrier()
        pltpu.sync_copy(
            acc.at[pl.ds(s * rows_per_tile, rows_per_tile)],
            out.at[c, pl.ds(s * rows_per_tile, rows_per_tile)],
        )

    return k


def _bn_in(x, gamma, beta):
    m = jnp.mean(x, axis=0, keepdims=True)
    v = jnp.mean((x - m) * (x - m), axis=0, keepdims=True)
    return (x - m) / jnp.sqrt(v + 1e-5) * gamma + beta


def _dot(a, b):
    return jnp.dot(a, b, preferred_element_type=jnp.float32)


def _tc1_body(n, feats, W1, resW1, resb1, h_o, res1_o):
    f = feats[...]
    h_o[0:n, :] = _dot(f, W1[...])
    res1_o[...] = jnp.maximum(_dot(f, resW1[...]) + resb1[...], 0.0)


def _tc2_body(n, acc_rows, p, res1, b1, g1, be1, resW2, resb2, x1_o, res2_o):
    agg = p[0, :n, :] + p[1, :n, :]
    conv = jnp.maximum(agg + b1[...], 0.0)
    x1 = _bn_in(conv + res1[...], g1[...], be1[...])
    x1_o[0:n, :] = x1
    res2_o[...] = jnp.maximum(_dot(x1, resW2[...]) + resb2[...], 0.0)


def _tc3_body(n, nb, npad, p, res2, W2, b2, g2, be2, aw, ab, idsc, idsr, addf,
              M1, mb1, gm, bm, M2, mb2, out_o, x2s):
    agg2 = p[0, :n, :] + p[1, :n, :]
    conv2 = jnp.maximum(_dot(agg2, W2[...]) + b2[...], 0.0)
    x2 = _bn_in(conv2 + res2[...], g2[...], be2[...])
    feat = x2.shape[1]
    z = _dot(x2, aw[...]) + ab[...]
    wgt = 1.0 / (1.0 + jnp.exp(-z))
    gcol = lax.broadcasted_iota(jnp.int32, (nb, 1), 0)
    onehot_t = (gcol == idsr[...]).astype(jnp.float32)
    hsum = _dot(onehot_t, x2 * wgt)
    x2s[0:n, :] = x2
    if npad > n:
        x2s[n:npad, :] = jnp.zeros((npad - n, feat), jnp.float32)
    blk = 128
    gids = lax.broadcasted_iota(jnp.int32, (1, nb), 1)
    rpos = lax.broadcasted_iota(jnp.int32, (blk, 1), 0)
    neg = jnp.float32(-jnp.inf)

    def step(i, hmax):
        st = i * blk
        rows = x2s[pl.ds(st, blk), :]
        idc = idsc[pl.ds(st, blk), :]
        for s in (1, 2, 4, 8, 16, 32, 64):
            rsh = jnp.concatenate(
                [jnp.full((s, feat), neg), rows[: blk - s]], axis=0)
            ish = jnp.concatenate(
                [jnp.full((s, 1), -1, jnp.int32), idc[: blk - s]], axis=0)
            rows = jnp.where(idc == ish, jnp.maximum(rows, rsh), rows)
        idn = jnp.concatenate(
            [idc[1:], jnp.full((1, 1), -1, jnp.int32)], axis=0)
        is_end = (idc != idn) | (rpos == blk - 1)
        sel = ((idc == gids) & is_end).astype(jnp.float32)
        csum = lax.dot_general(sel, rows, (((0,), (0,)), ((), ())),
                               preferred_element_type=jnp.float32)
        cnt = lax.dot_general(sel, jnp.ones((blk, 1), jnp.float32),
                              (((0,), (0,)), ((), ())),
                              preferred_element_type=jnp.float32)
        cmax = jnp.where(cnt > 0.0, csum, neg)
        return jnp.maximum(hmax, cmax)

    hmax = lax.fori_loop(0, npad // blk, step,
                         jnp.full((nb, feat), neg, jnp.float32))
    gfeat = jnp.concatenate([hsum, hmax, addf[...]], axis=1)
    hmlp = jnp.maximum(_dot(gfeat, M1[...]) + mb1[...], 0.0)
    hmlp = _bn_in(hmlp, gm[...], bm[...])
    out_o[...] = _dot(hmlp, M2[...]) + mb2[...]


def kernel(feats, edge_index, node_to_graph, add_feats, W1, b1, resW1, resb1,
           g1, be1, W2, b2, resW2, resb2, g2, be2, aw, ab, M1, mb1, gm, bm,
           M2, mb2):
    n, d = feats.shape
    h = W1.shape[1]
    nb = add_feats.shape[0]
    e = edge_index.shape[1]

    nchunk = 2 * _ceil_div(e, _NW * _CK * 2)
    e_pad = _NW * _CK * nchunk
    acc_rows = _ceil_div(n + 1, _NS * 8) * _NS * 8
    rows_per_tile = acc_rows // _NS

    src, dst = edge_index[0], edge_index[1]
    src3 = jnp.concatenate(
        [src, jnp.zeros((e_pad - e,), jnp.int32)]).reshape(_NW, nchunk, _CK)
    src3 = jnp.concatenate(
        [src3, jnp.zeros((_NW, 1, _CK), jnp.int32)], axis=1)
    dst3 = jnp.concatenate(
        [dst, jnp.full((e_pad - e,), n, jnp.int32)]).reshape(_NW, nchunk, _CK)
    dst3 = jnp.concatenate(
        [dst3, jnp.full((_NW, 1, _CK), n, jnp.int32)], axis=1)
    zrows = jnp.zeros((rows_per_tile, h), jnp.float32)

    scatter = _make_edge_scatter_add(n, h, nchunk, acc_rows)

    r = lambda v: v.reshape(1, -1)
    npad = _ceil_div(n, 128) * 128
    idsc = jnp.concatenate(
        [node_to_graph, jnp.full((npad - n,), -1, jnp.int32)]).reshape(npad, 1)
    idsr = node_to_graph.reshape(1, n)

    h1, res1 = pl.pallas_call(
        functools.partial(_tc1_body, n),
        out_shape=[jax.ShapeDtypeStruct((acc_rows, h), jnp.float32),
                   jax.ShapeDtypeStruct((n, h), jnp.float32)],
    )(feats, W1, resW1, r(resb1))

    p1 = scatter(h1, src3, dst3, zrows)

    x1, res2 = pl.pallas_call(
        functools.partial(_tc2_body, n, acc_rows),
        out_shape=[jax.ShapeDtypeStruct((acc_rows, h), jnp.float32),
                   jax.ShapeDtypeStruct((n, h), jnp.float32)],
    )(p1, res1, r(b1), r(g1), r(be1), resW2, r(resb2))

    p2 = scatter(x1, src3, dst3, zrows)

    out = pl.pallas_call(
        functools.partial(_tc3_body, n, nb, npad),
        out_shape=jax.ShapeDtypeStruct((nb, M2.shape[1]), jnp.float32),
        scratch_shapes=[pltpu.VMEM((npad, h), jnp.float32)],
    )(p2, res2, W2, r(b2), r(g2), r(be2), aw, r(ab), idsc, idsr, add_feats,
      M1, r(mb1), r(gm), r(bm), M2, r(mb2))
    return out


# speedup vs baseline: 2.4507x; 1.1528x over previous
import functools

import jax
import jax.numpy as jnp
from jax import lax
from jax.experimental import pallas as pl
from jax.experimental.pallas import tpu as pltpu
from jax.experimental.pallas import tpu_sc as plsc

_NC = 2
_NS = 16
_NW = _NC * _NS
_CK = 400


def _ceil_div(a, b):
    return (a + b - 1) // b


def _make_edge_scatter_add(n_rows, feat, nchunk, acc_rows):
    rows_per_tile = acc_rows // _NS
    mesh = plsc.VectorSubcoreMesh(core_axis_name="c", subcore_axis_name="s")

    @functools.partial(
        pl.kernel,
        mesh=mesh,
        out_type=jax.ShapeDtypeStruct((_NC, acc_rows, feat), jnp.float32),
        scratch_types=[
            pltpu.VMEM((nchunk, _CK), jnp.int32),
            pltpu.VMEM((nchunk, _CK), jnp.int32),
            pltpu.VMEM((_CK, feat), jnp.float32),
            pltpu.VMEM_SHARED((acc_rows, feat), jnp.float32),
            pltpu.VMEM_SHARED((acc_rows, feat), jnp.float32),
            pltpu.SemaphoreType.DMA,
        ],
        compiler_params=pltpu.CompilerParams(use_tc_tiling_on_sc=False),
    )
    def k(table, src3, dst3, zrows, out, src_v, dst_v, rows0, acc, tsp, sem0):
        c = lax.axis_index("c")
        s = lax.axis_index("s")
        wid = s * _NC + c
        pltpu.sync_copy(zrows, acc.at[pl.ds(s * rows_per_tile, rows_per_tile)])
        pltpu.sync_copy(table.at[pl.ds(s * rows_per_tile, rows_per_tile)],
                        tsp.at[pl.ds(s * rows_per_tile, rows_per_tile)])
        pltpu.sync_copy(src3.at[wid], src_v)
        pltpu.sync_copy(dst3.at[wid], dst_v)
        plsc.subcore_barrier()

        def chunk(j, carry):
            pltpu.async_copy(tsp.at[src_v.at[j]], rows0, sem0).wait()
            pltpu.sync_copy(rows0, acc.at[dst_v.at[j]], add=True)
            return carry

        lax.fori_loop(0, nchunk, chunk, 0)
        plsc.subcore_barrier()
        pltpu.sync_copy(
            acc.at[pl.ds(s * rows_per_tile, rows_per_tile)],
            out.at[c, pl.ds(s * rows_per_tile, rows_per_tile)],
        )

    return k


def _bn_in(x, gamma, beta):
    m = jnp.mean(x, axis=0, keepdims=True)
    v = jnp.mean((x - m) * (x - m), axis=0, keepdims=True)
    return (x - m) / jnp.sqrt(v + 1e-5) * gamma + beta


def _dot(a, b):
    return jnp.dot(a, b, preferred_element_type=jnp.float32)


def _tc1_body(n, feats, W1, resW1, resb1, h_o, res1_o):
    f = feats[...]
    h_o[0:n, :] = _dot(f, W1[...])
    res1_o[...] = jnp.maximum(_dot(f, resW1[...]) + resb1[...], 0.0)


def _tc2_body(n, acc_rows, p, res1, b1, g1, be1, resW2, resb2, x1_o, res2_o):
    agg = p[0, :n, :] + p[1, :n, :]
    conv = jnp.maximum(agg + b1[...], 0.0)
    x1 = _bn_in(conv + res1[...], g1[...], be1[...])
    x1_o[0:n, :] = x1
    res2_o[...] = jnp.maximum(_dot(x1, resW2[...]) + resb2[...], 0.0)


def _tc3_body(n, nb, npad, p, res2, W2, b2, g2, be2, aw, ab, idsc, idsr, addf,
              M1, mb1, gm, bm, M2, mb2, out_o, x2s):
    agg2 = p[0, :n, :] + p[1, :n, :]
    conv2 = jnp.maximum(_dot(agg2, W2[...]) + b2[...], 0.0)
    x2 = _bn_in(conv2 + res2[...], g2[...], be2[...])
    feat = x2.shape[1]
    z = _dot(x2, aw[...]) + ab[...]
    wgt = 1.0 / (1.0 + jnp.exp(-z))
    gcol = lax.broadcasted_iota(jnp.int32, (nb, 1), 0)
    onehot_t = (gcol == idsr[...]).astype(jnp.float32)
    hsum = _dot(onehot_t, x2 * wgt)
    x2s[0:n, :] = x2
    if npad > n:
        x2s[n:npad, :] = jnp.zeros((npad - n, feat), jnp.float32)
    blk = 128
    gids = lax.broadcasted_iota(jnp.int32, (1, nb), 1)
    rpos = lax.broadcasted_iota(jnp.int32, (blk, 1), 0)
    neg = jnp.float32(-jnp.inf)

    def step(i, hmax):
        st = i * blk
        rows = x2s[pl.ds(st, blk), :]
        idc = idsc[pl.ds(st, blk), :]
        for s in (1, 2, 4, 8, 16, 32, 64):
            rsh = jnp.concatenate(
                [jnp.full((s, feat), neg), rows[: blk - s]], axis=0)
            ish = jnp.concatenate(
                [jnp.full((s, 1), -1, jnp.int32), idc[: blk - s]], axis=0)
            rows = jnp.where(idc == ish, jnp.maximum(rows, rsh), rows)
        idn = jnp.concatenate(
            [idc[1:], jnp.full((1, 1), -1, jnp.int32)], axis=0)
        is_end = (idc != idn) | (rpos == blk - 1)
        sel = ((idc == gids) & is_end).astype(jnp.float32)
        csum = lax.dot_general(sel, rows, (((0,), (0,)), ((), ())),
                               preferred_element_type=jnp.float32)
        cnt = lax.dot_general(sel, jnp.ones((blk, 1), jnp.float32),
                              (((0,), (0,)), ((), ())),
                              preferred_element_type=jnp.float32)
        cmax = jnp.where(cnt > 0.0, csum, neg)
        return jnp.maximum(hmax, cmax)

    hmax = lax.fori_loop(0, npad // blk, step,
                         jnp.full((nb, feat), neg, jnp.float32))
    gfeat = jnp.concatenate([hsum, hmax, addf[...]], axis=1)
    hmlp = jnp.maximum(_dot(gfeat, M1[...]) + mb1[...], 0.0)
    hmlp = _bn_in(hmlp, gm[...], bm[...])
    out_o[...] = _dot(hmlp, M2[...]) + mb2[...]


def kernel(feats, edge_index, node_to_graph, add_feats, W1, b1, resW1, resb1,
           g1, be1, W2, b2, resW2, resb2, g2, be2, aw, ab, M1, mb1, gm, bm,
           M2, mb2):
    n, d = feats.shape
    h = W1.shape[1]
    nb = add_feats.shape[0]
    e = edge_index.shape[1]

    nchunk = _ceil_div(e, _NW * _CK)
    e_pad = _NW * _CK * nchunk
    acc_rows = _ceil_div(n + 1, _NS * 8) * _NS * 8
    rows_per_tile = acc_rows // _NS

    src, dst = edge_index[0], edge_index[1]
    if e_pad > e:
        src = jnp.concatenate([src, jnp.zeros((e_pad - e,), jnp.int32)])
        dst = jnp.concatenate([dst, jnp.full((e_pad - e,), n, jnp.int32)])
    src3 = src.reshape(_NW, nchunk, _CK)
    dst3 = dst.reshape(_NW, nchunk, _CK)
    zrows = jnp.zeros((rows_per_tile, h), jnp.float32)

    scatter = _make_edge_scatter_add(n, h, nchunk, acc_rows)

    r = lambda v: v.reshape(1, -1)
    npad = _ceil_div(n, 128) * 128
    idsc = jnp.concatenate(
        [node_to_graph, jnp.full((npad - n,), -1, jnp.int32)]).reshape(npad, 1)
    idsr = node_to_graph.reshape(1, n)

    h1, res1 = pl.pallas_call(
        functools.partial(_tc1_body, n),
        out_shape=[jax.ShapeDtypeStruct((acc_rows, h), jnp.float32),
                   jax.ShapeDtypeStruct((n, h), jnp.float32)],
    )(feats, W1, resW1, r(resb1))

    p1 = scatter(h1, src3, dst3, zrows)

    x1, res2 = pl.pallas_call(
        functools.partial(_tc2_body, n, acc_rows),
        out_shape=[jax.ShapeDtypeStruct((acc_rows, h), jnp.float32),
                   jax.ShapeDtypeStruct((n, h), jnp.float32)],
    )(p1, res1, r(b1), r(g1), r(be1), resW2, r(resb2))

    p2 = scatter(x1, src3, dst3, zrows)

    out = pl.pallas_call(
        functools.partial(_tc3_body, n, nb, npad),
        out_shape=jax.ShapeDtypeStruct((nb, M2.shape[1]), jnp.float32),
        scratch_shapes=[pltpu.VMEM((npad, h), jnp.float32)],
    )(p2, res2, W2, r(b2), r(g2), r(be2), aw, r(ab), idsc, idsr, add_feats,
      M1, r(mb1), r(gm), r(bm), M2, r(mb2))
    return out
